# baseline (speedup 1.0000x reference)
import functools

import jax
import jax.numpy as jnp
from jax import lax
from jax.experimental import pallas as pl
from jax.experimental.pallas import tpu as pltpu
from jax.experimental.pallas import tpu_sc as plsc

VOCAB = 100000
EMBED_DIM = 16
BATCH = 16384

_INFO = plsc.get_sparse_core_info()
_NC = _INFO.num_cores
_NS = _INFO.num_subcores
_L = _INFO.num_lanes
_NW = _NC * _NS
_B_PER_W = BATCH // _NW

_MESH = plsc.VectorSubcoreMesh(core_axis_name="c", subcore_axis_name="s")


@functools.partial(
    pl.kernel,
    mesh=_MESH,
    out_type=jax.ShapeDtypeStruct((BATCH, EMBED_DIM), jnp.float32),
    scratch_types=[
        pltpu.VMEM((_B_PER_W,), jnp.int32),
        pltpu.VMEM((_B_PER_W, EMBED_DIM), jnp.float32),
        pltpu.SemaphoreType.DMA,
    ],
    compiler_params=pltpu.CompilerParams(use_tc_tiling_on_sc=False),
)
def _embed_gather(idx_hbm, table_hbm, out_hbm, idx_v, rows_v, sem):
    wid = lax.axis_index("s") * _NC + lax.axis_index("c")
    base = wid * _B_PER_W
    pltpu.sync_copy(idx_hbm.at[pl.ds(base, _B_PER_W)], idx_v)
    for i in range(_B_PER_W // _L):
        sl = pl.ds(i * _L, _L)
        idx_v[sl] = idx_v[sl] + 1
    pltpu.async_copy(table_hbm.at[idx_v], rows_v, sem).wait()
    pltpu.sync_copy(rows_v, out_hbm.at[pl.ds(base, _B_PER_W)])


def kernel(source_id, table):
    return _embed_gather(source_id.astype(jnp.int32), table)


# speedup vs baseline: 1.6901x; 1.6901x over previous
import functools

import jax
import jax.numpy as jnp
from jax import lax
from jax.experimental import pallas as pl
from jax.experimental.pallas import tpu as pltpu
from jax.experimental.pallas import tpu_sc as plsc

VOCAB = 100000
EMBED_DIM = 16
BATCH = 16384
VOCAB_PAD = 100008

_INFO = plsc.get_sparse_core_info()
_NC = _INFO.num_cores
_NS = _INFO.num_subcores
_L = _INFO.num_lanes
_B_HALF = BATCH // _NC

_MESH = plsc.VectorSubcoreMesh(core_axis_name="c", subcore_axis_name="s")


@functools.partial(
    pl.kernel,
    mesh=_MESH,
    out_type=jax.ShapeDtypeStruct((EMBED_DIM, BATCH), jnp.float32),
    scratch_types=[
        pltpu.VMEM((VOCAB_PAD,), jnp.float32),
        pltpu.VMEM((_B_HALF,), jnp.int32),
        pltpu.VMEM((_B_HALF,), jnp.float32),
        pltpu.SemaphoreType.DMA,
        pltpu.SemaphoreType.DMA,
    ],
    compiler_params=pltpu.CompilerParams(
        use_tc_tiling_on_sc=False, needs_layout_passes=False
    ),
)
def _embed_gather(idx_hbm, tableT_hbm, outT_hbm, slab_v, idx_v, out_v, sem_a, sem_b):
    c = lax.axis_index("c")
    s = lax.axis_index("s")
    cp_slab = pltpu.async_copy(tableT_hbm.at[s], slab_v, sem_a)
    cp_idx = pltpu.async_copy(idx_hbm.at[pl.ds(c * _B_HALF, _B_HALF)], idx_v, sem_b)
    cp_idx.wait()
    cp_slab.wait()

    def body(i, carry):
        sl = pl.ds(i * _L, _L)
        iv = idx_v[sl] + 1
        out_v[sl] = plsc.load_gather(slab_v, [iv])
        return carry

    lax.fori_loop(0, _B_HALF // _L, body, 0, unroll=8)
    pltpu.sync_copy(out_v, outT_hbm.at[s, pl.ds(c * _B_HALF, _B_HALF)])


def kernel(source_id, table):
    tableT = jnp.pad(table.T, ((0, 0), (0, VOCAB_PAD - (VOCAB + 1))))
    outT = _embed_gather(source_id.astype(jnp.int32), tableT)
    return outT.T


# speedup vs baseline: 2.4918x; 1.4743x over previous
import functools

import jax
import jax.numpy as jnp
from jax import lax
from jax.experimental import pallas as pl
from jax.experimental.pallas import tpu as pltpu
from jax.experimental.pallas import tpu_sc as plsc

VOCAB = 100000
EMBED_DIM = 16
BATCH = 16384

_INFO = plsc.get_sparse_core_info()
_NC = _INFO.num_cores
_NS = _INFO.num_subcores
_L = _INFO.num_lanes
_B_HALF = BATCH // _NC

_MESH = plsc.VectorSubcoreMesh(core_axis_name="c", subcore_axis_name="s")


@functools.partial(
    pl.kernel,
    mesh=_MESH,
    out_type=jax.ShapeDtypeStruct((EMBED_DIM, BATCH), jnp.float32),
    scratch_types=[
        pltpu.VMEM((VOCAB + 1,), jnp.float32),
        pltpu.VMEM((_B_HALF,), jnp.int32),
        pltpu.VMEM((_B_HALF,), jnp.float32),
        pltpu.SemaphoreType.DMA,
        pltpu.SemaphoreType.DMA,
    ],
    compiler_params=pltpu.CompilerParams(
        use_tc_tiling_on_sc=True, needs_layout_passes=False
    ),
)
def _embed_gather(idx_hbm, tableT_hbm, outT_hbm, slab_v, idx_v, out_v, sem_a, sem_b):
    c = lax.axis_index("c")
    s = lax.axis_index("s")
    cp_slab = pltpu.async_copy(tableT_hbm.at[s], slab_v, sem_a)
    cp_idx = pltpu.async_copy(idx_hbm.at[pl.ds(c * _B_HALF, _B_HALF)], idx_v, sem_b)
    cp_idx.wait()
    cp_slab.wait()

    def body(i, carry):
        sl = pl.ds(i * _L, _L)
        iv = idx_v[sl] + 1
        out_v[sl] = plsc.load_gather(slab_v, [iv])
        return carry

    lax.fori_loop(0, _B_HALF // _L, body, 0, unroll=8)
    pltpu.sync_copy(out_v, outT_hbm.at[s, pl.ds(c * _B_HALF, _B_HALF)])


def kernel(source_id, table):
    outT = _embed_gather(source_id.astype(jnp.int32), table.T)
    return outT.T
